# baseline (speedup 1.0000x reference)
import functools

import jax
import jax.numpy as jnp
from jax import lax
from jax.experimental import pallas as pl
from jax.experimental.pallas import tpu as pltpu
from jax.experimental.pallas import tpu_sc as plsc

D_MODEL = 64
NUM_CORES = 2
NUM_SUBCORES = 16
NW = NUM_CORES * NUM_SUBCORES
V = 1000000
NSPLIT = 4
VSLICE = V // NSPLIT


@functools.lru_cache(maxsize=None)
def _make_gather(B: int, chunk: int):
    assert B % (NW * chunk) == 0
    b_per_w = B // NW
    n_chunks = b_per_w // chunk
    mesh = plsc.VectorSubcoreMesh(
        core_axis_name="c", subcore_axis_name="s",
        num_cores=NUM_CORES, num_subcores=NUM_SUBCORES)

    @functools.partial(
        pl.kernel,
        out_type=jax.ShapeDtypeStruct((B, D_MODEL), jnp.float32),
        mesh=mesh,
        compiler_params=pltpu.CompilerParams(use_tc_tiling_on_sc=False),
        scratch_types=[
            pltpu.VMEM((b_per_w,), jnp.int32),
            [pltpu.VMEM((NSPLIT, chunk), jnp.int32) for _ in range(2)],
            [pltpu.VMEM((chunk, D_MODEL), jnp.float32) for _ in range(2)],
            [pltpu.SemaphoreType.DMA for _ in range(2)],
            [pltpu.SemaphoreType.DMA for _ in range(2)],
        ],
    )
    def k(idx_hbm, *args):
        tables = args[:NSPLIT]
        out_hbm = args[NSPLIT]
        idx_v, rid, rows, gsem, wsem = args[NSPLIT + 1:]
        wid = lax.axis_index("s") * NUM_CORES + lax.axis_index("c")
        base = wid * b_per_w

        pltpu.sync_copy(idx_hbm.at[pl.ds(base, b_per_w)], idx_v)

        def start_gather(c, b):
            def rid_body(i, carry):
                toks = idx_v[pl.ds(c * chunk + i * 16, 16)]
                for t in range(NSPLIT):
                    rel = toks - t * VSLICE
                    in_slice = (toks >= t * VSLICE) & (toks < (t + 1) * VSLICE)
                    rid[b][t, pl.ds(i * 16, 16)] = jnp.where(in_slice, rel, -1)
                return carry
            lax.fori_loop(0, chunk // 16, rid_body, 0, unroll=True)
            return [
                pltpu.async_copy(
                    tables[t].at[plsc.Indices(rid[b].at[t], ignored_value=-1)],
                    rows[b], gsem[b])
                for t in range(NSPLIT)
            ]

        gh = [None, None]
        wh = [None, None]
        for c in range(min(2, n_chunks)):
            gh[c] = start_gather(c, c)
        for c in range(n_chunks):
            b = c & 1
            for h in gh[b]:
                h.wait()
            wh[b] = pltpu.async_copy(
                rows[b], out_hbm.at[pl.ds(base + c * chunk, chunk)], wsem[b])
            if c + 2 < n_chunks:
                wh[b].wait()
                gh[b] = start_gather(c + 2, b)
        for b in range(min(2, n_chunks)):
            if wh[b] is not None:
                wh[b].wait()

    return k


def kernel(tokens, W_E):
    B = tokens.size
    flat = tokens.reshape(-1).astype(jnp.int32)
    slices = [W_E[t * VSLICE:(t + 1) * VSLICE] for t in range(NSPLIT)]
    out = _make_gather(B, 640)(flat, *slices)
    return out.reshape(*tokens.shape, D_MODEL)


# speedup vs baseline: 1.4826x; 1.4826x over previous
import functools

import jax
import jax.numpy as jnp
from jax import lax
from jax.experimental import pallas as pl
from jax.experimental.pallas import tpu as pltpu
from jax.experimental.pallas import tpu_sc as plsc

D_MODEL = 64
PAD_W = 128
NUM_CORES = 2
NUM_SUBCORES = 16
NW = NUM_CORES * NUM_SUBCORES


@functools.lru_cache(maxsize=None)
def _make_gather(B: int, V: int, chunk: int):
    assert B % (NW * chunk) == 0
    b_per_w = B // NW
    n_chunks = b_per_w // chunk
    mesh = plsc.VectorSubcoreMesh(
        core_axis_name="c", subcore_axis_name="s",
        num_cores=NUM_CORES, num_subcores=NUM_SUBCORES)

    @functools.partial(
        pl.kernel,
        out_type=jax.ShapeDtypeStruct((B, D_MODEL), jnp.float32),
        mesh=mesh,
        compiler_params=pltpu.CompilerParams(use_tc_tiling_on_sc=False),
        scratch_types=[
            pltpu.VMEM((b_per_w,), jnp.int32),
            pltpu.VMEM((chunk, PAD_W), jnp.float32),
            pltpu.VMEM((chunk, PAD_W), jnp.float32),
            pltpu.SemaphoreType.DMA,
            pltpu.SemaphoreType.DMA,
            pltpu.SemaphoreType.DMA,
            pltpu.SemaphoreType.DMA,
        ],
    )
    def k(idx_hbm, table_hbm, out_hbm,
          idx_v, rows0, rows1, g0, g1, w0, w1):
        wid = lax.axis_index("s") * NUM_CORES + lax.axis_index("c")
        base = wid * b_per_w
        rows = [rows0, rows1]
        gsem = [g0, g1]
        wsem = [w0, w1]

        pltpu.sync_copy(idx_hbm.at[pl.ds(base, b_per_w)], idx_v)

        def start_gather(c, b):
            return pltpu.async_copy(
                table_hbm.at[idx_v.at[pl.ds(c * chunk, chunk)]],
                rows[b], gsem[b])

        gh = [None, None]
        wh = [None, None]
        for c in range(min(2, n_chunks)):
            gh[c] = start_gather(c, c)
        for c in range(n_chunks):
            b = c & 1
            gh[b].wait()
            wh[b] = pltpu.async_copy(
                rows[b].at[:, pl.ds(0, D_MODEL)],
                out_hbm.at[pl.ds(base + c * chunk, chunk)], wsem[b])
            if c + 2 < n_chunks:
                wh[b].wait()
                gh[b] = start_gather(c + 2, b)
        for b in range(min(2, n_chunks)):
            if wh[b] is not None:
                wh[b].wait()

    return k


def kernel(tokens, W_E):
    B = tokens.size
    V = W_E.shape[0]
    flat = tokens.reshape(-1).astype(jnp.int32)
    padded = jnp.pad(W_E, ((0, 0), (0, PAD_W - D_MODEL)))
    out = _make_gather(B, V, 400)(flat, padded)
    return out.reshape(*tokens.shape, D_MODEL)


# speedup vs baseline: 1.6237x; 1.0952x over previous
import functools

import jax
import jax.numpy as jnp
from jax import lax
from jax.experimental import pallas as pl
from jax.experimental.pallas import tpu as pltpu
from jax.experimental.pallas import tpu_sc as plsc

D_MODEL = 64
PAD_W = 128
NUM_CORES = 2
NUM_SUBCORES = 16
NW = NUM_CORES * NUM_SUBCORES


@functools.lru_cache(maxsize=None)
def _make_gather(B: int, V: int, chunk: int):
    assert B % (NW * chunk) == 0
    b_per_w = B // NW
    n_chunks = b_per_w // chunk
    mesh = plsc.VectorSubcoreMesh(
        core_axis_name="c", subcore_axis_name="s",
        num_cores=NUM_CORES, num_subcores=NUM_SUBCORES)

    @functools.partial(
        pl.kernel,
        out_type=jax.ShapeDtypeStruct((B, D_MODEL), jnp.float32),
        mesh=mesh,
        compiler_params=pltpu.CompilerParams(use_tc_tiling_on_sc=False),
        scratch_types=[
            pltpu.VMEM((b_per_w,), jnp.int32),
            pltpu.VMEM((chunk, PAD_W), jnp.float32),
            pltpu.VMEM((chunk, PAD_W), jnp.float32),
            pltpu.SemaphoreType.DMA,
            pltpu.SemaphoreType.DMA,
            pltpu.SemaphoreType.DMA,
            pltpu.SemaphoreType.DMA,
        ],
    )
    def k(idx_hbm, table_hbm, out_hbm,
          idx_v, rows0, rows1, g0, g1, w0, w1):
        wid = lax.axis_index("s") * NUM_CORES + lax.axis_index("c")
        base = wid * b_per_w
        rows = [rows0, rows1]
        gsem = [g0, g1]
        wsem = [w0, w1]

        pltpu.sync_copy(idx_hbm.at[pl.ds(base, b_per_w)], idx_v)

        def start_gather(c, b):
            return pltpu.async_copy(
                table_hbm.at[idx_v.at[pl.ds(c * chunk, chunk)]],
                rows[b], gsem[b])

        gh = [None, None]
        wh = [None, None]
        for c in range(min(2, n_chunks)):
            gh[c] = start_gather(c, c)
        for c in range(n_chunks):
            b = c & 1
            gh[b].wait()
            wh[b] = pltpu.async_copy(
                rows[b].at[:, pl.ds(0, D_MODEL)],
                out_hbm.at[pl.ds(base + c * chunk, chunk)], wsem[b])
            if c + 2 < n_chunks:
                wh[b].wait()
                gh[b] = start_gather(c + 2, b)
        for b in range(min(2, n_chunks)):
            if wh[b] is not None:
                wh[b].wait()

    return k


@functools.lru_cache(maxsize=None)
def _make_transpose(V: int, blk: int):
    grid = (V + blk - 1) // blk

    def body(x_ref, o_ref):
        o_ref[:, :D_MODEL] = x_ref[...].T
        o_ref[:, D_MODEL:] = jnp.zeros((blk, PAD_W - D_MODEL), jnp.float32)

    return pl.pallas_call(
        body,
        grid=(grid,),
        in_specs=[pl.BlockSpec((D_MODEL, blk), lambda i: (0, i))],
        out_specs=pl.BlockSpec((blk, PAD_W), lambda i: (i, 0)),
        out_shape=jax.ShapeDtypeStruct((V, PAD_W), jnp.float32),
    )


def kernel(tokens, W_E):
    B = tokens.size
    V = W_E.shape[0]
    flat = tokens.reshape(-1).astype(jnp.int32)
    padded = _make_transpose(V, 2048)(W_E.T)
    out = _make_gather(B, V, 400)(flat, padded)
    return out.reshape(*tokens.shape, D_MODEL)


# speedup vs baseline: 2.3068x; 1.4207x over previous
import functools

import jax
import jax.numpy as jnp
from jax import lax
from jax.experimental import pallas as pl
from jax.experimental.pallas import tpu as pltpu
from jax.experimental.pallas import tpu_sc as plsc

D_MODEL = 64
PAD_W = 128
NUM_CORES = 2
NUM_SUBCORES = 16
NW = NUM_CORES * NUM_SUBCORES


@functools.lru_cache(maxsize=None)
def _make_gather(B: int, V: int, chunk: int):
    assert B % (NW * chunk) == 0
    b_per_w = B // NW
    n_chunks = b_per_w // chunk
    mesh = plsc.VectorSubcoreMesh(
        core_axis_name="c", subcore_axis_name="s",
        num_cores=NUM_CORES, num_subcores=NUM_SUBCORES)

    @functools.partial(
        pl.kernel,
        out_type=jax.ShapeDtypeStruct((B, D_MODEL), jnp.float32),
        mesh=mesh,
        compiler_params=pltpu.CompilerParams(use_tc_tiling_on_sc=False),
        scratch_types=[
            pltpu.VMEM((b_per_w,), jnp.int32),
            pltpu.VMEM((chunk, PAD_W), jnp.float32),
            pltpu.VMEM((chunk, PAD_W), jnp.float32),
            pltpu.SemaphoreType.DMA,
            pltpu.SemaphoreType.DMA,
            pltpu.SemaphoreType.DMA,
            pltpu.SemaphoreType.DMA,
        ],
    )
    def k(idx_hbm, table_hbm, out_hbm,
          idx_v, rows0, rows1, g0, g1, w0, w1):
        wid = lax.axis_index("s") * NUM_CORES + lax.axis_index("c")
        base = wid * b_per_w
        rows = [rows0, rows1]
        gsem = [g0, g1]
        wsem = [w0, w1]

        pltpu.sync_copy(idx_hbm.at[pl.ds(base, b_per_w)], idx_v)

        def start_gather(c, b):
            return pltpu.async_copy(
                table_hbm.at[idx_v.at[pl.ds(c * chunk, chunk)]],
                rows[b], gsem[b])

        gh = [None, None]
        wh = [None, None]
        for c in range(min(2, n_chunks)):
            gh[c] = start_gather(c, c)
        for c in range(n_chunks):
            b = c & 1
            gh[b].wait()
            wh[b] = pltpu.async_copy(
                rows[b].at[:, pl.ds(0, D_MODEL)],
                out_hbm.at[pl.ds(base + c * chunk, chunk)], wsem[b])
            if c + 2 < n_chunks:
                wh[b].wait()
                gh[b] = start_gather(c + 2, b)
        for b in range(min(2, n_chunks)):
            if wh[b] is not None:
                wh[b].wait()

    return k


@functools.lru_cache(maxsize=None)
def _make_transpose(V: int, blk: int):
    grid = (V + blk - 1) // blk

    def body(x_ref, o_ref):
        o_ref[:, :D_MODEL] = x_ref[...].T
        o_ref[:, D_MODEL:] = jnp.zeros((blk, PAD_W - D_MODEL), jnp.float32)

    return pl.pallas_call(
        body,
        grid=(grid,),
        in_specs=[pl.BlockSpec((D_MODEL, blk), lambda i: (0, i))],
        out_specs=pl.BlockSpec((blk, PAD_W), lambda i: (i, 0)),
        out_shape=jax.ShapeDtypeStruct((V, PAD_W), jnp.float32),
    )


def kernel(tokens, W_E):
    B = tokens.size
    V = W_E.shape[0]
    flat = tokens.reshape(-1).astype(jnp.int32)
    padded = _make_transpose(V, 8192)(W_E.T)
    out = _make_gather(B, V, 400)(flat, padded)
    return out.reshape(*tokens.shape, D_MODEL)


# speedup vs baseline: 2.4035x; 1.0419x over previous
import functools

import jax
import jax.numpy as jnp
from jax import lax
from jax.experimental import pallas as pl
from jax.experimental.pallas import tpu as pltpu
from jax.experimental.pallas import tpu_sc as plsc

D_MODEL = 64
PAD_W = 128
NUM_CORES = 2
NUM_SUBCORES = 16
NW = NUM_CORES * NUM_SUBCORES


@functools.lru_cache(maxsize=None)
def _make_gather(B: int, V: int, chunk: int):
    assert B % (NW * chunk) == 0
    b_per_w = B // NW
    n_chunks = b_per_w // chunk
    mesh = plsc.VectorSubcoreMesh(
        core_axis_name="c", subcore_axis_name="s",
        num_cores=NUM_CORES, num_subcores=NUM_SUBCORES)

    @functools.partial(
        pl.kernel,
        out_type=jax.ShapeDtypeStruct((B, D_MODEL), jnp.float32),
        mesh=mesh,
        compiler_params=pltpu.CompilerParams(use_tc_tiling_on_sc=False),
        scratch_types=[
            pltpu.VMEM((b_per_w,), jnp.int32),
            pltpu.VMEM((chunk, PAD_W), jnp.float32),
            pltpu.VMEM((chunk, PAD_W), jnp.float32),
            pltpu.SemaphoreType.DMA,
            pltpu.SemaphoreType.DMA,
            pltpu.SemaphoreType.DMA,
            pltpu.SemaphoreType.DMA,
        ],
    )
    def k(idx_hbm, table_hbm, out_hbm,
          idx_v, rows0, rows1, g0, g1, w0, w1):
        wid = lax.axis_index("s") * NUM_CORES + lax.axis_index("c")
        base = wid * b_per_w
        rows = [rows0, rows1]
        gsem = [g0, g1]
        wsem = [w0, w1]

        pltpu.sync_copy(idx_hbm.at[pl.ds(base, b_per_w)], idx_v)

        def start_gather(c, b):
            return pltpu.async_copy(
                table_hbm.at[idx_v.at[pl.ds(c * chunk, chunk)]],
                rows[b], gsem[b])

        gh = [None, None]
        wh = [None, None]
        for c in range(min(2, n_chunks)):
            gh[c] = start_gather(c, c)
        for c in range(n_chunks):
            b = c & 1
            gh[b].wait()
            wh[b] = pltpu.async_copy(
                rows[b].at[:, pl.ds(0, D_MODEL)],
                out_hbm.at[pl.ds(base + c * chunk, chunk)], wsem[b])
            if c + 2 < n_chunks:
                wh[b].wait()
                gh[b] = start_gather(c + 2, b)
        for b in range(min(2, n_chunks)):
            if wh[b] is not None:
                wh[b].wait()

    return k


@functools.lru_cache(maxsize=None)
def _make_transpose(V: int, blk: int):
    grid = (V + blk - 1) // blk

    def body(x_ref, o_ref):
        o_ref[:, :D_MODEL] = x_ref[...].T
        o_ref[:, D_MODEL:] = jnp.zeros((blk, PAD_W - D_MODEL), jnp.float32)

    return pl.pallas_call(
        body,
        grid=(grid,),
        in_specs=[pl.BlockSpec((D_MODEL, blk), lambda i: (0, i))],
        out_specs=pl.BlockSpec((blk, PAD_W), lambda i: (i, 0)),
        out_shape=jax.ShapeDtypeStruct((V, PAD_W), jnp.float32),
    )


def kernel(tokens, W_E):
    B = tokens.size
    V = W_E.shape[0]
    flat = tokens.reshape(-1).astype(jnp.int32)
    padded = _make_transpose(V, 16384)(W_E.T)
    out = _make_gather(B, V, 400)(flat, padded)
    return out.reshape(*tokens.shape, D_MODEL)
